# baseline (speedup 1.0000x reference)
import functools

import jax
import jax.numpy as jnp
from jax import lax
from jax.experimental import pallas as pl
from jax.experimental.pallas import tpu as pltpu
from jax.experimental.pallas import tpu_sc as plsc

_B = 4
_MD = 128
_N = 8192
_H = 4
_F = 32
_S = 64
_NB = 2048
_NPC = 1024
_G = _S * _S
_CNT = _B * _N

_SC_PARAMS = pltpu.CompilerParams(use_tc_tiling_on_sc=False,
                                  needs_layout_passes=False)


def _sc_mesh():
    return plsc.VectorSubcoreMesh(core_axis_name="c", subcore_axis_name="s",
                                  num_cores=2, num_subcores=16)


def _stage_a_body(x_ref, wvt_ref, wk_ref, vt_ref, kvk_ref,
                  vs1_ref, vs2_ref, ks1_ref, ks2_ref):
    b = pl.program_id(0)
    nb = pl.program_id(1)
    x = x_ref[0]
    vtc = lax.dot_general(x, wvt_ref[...], (((0,), (0,)), ((), ())),
                          preferred_element_type=jnp.float32)
    vt_ref[0] = vtc
    kc = jnp.dot(wk_ref[...], x, preferred_element_type=jnp.float32)
    kvk_ref[0] = kc
    pv1 = jnp.sum(vtc, axis=0).reshape(1, 128)
    pv2 = jnp.sum(vtc * vtc, axis=0).reshape(1, 128)
    pk1 = jnp.sum(kc, axis=1).reshape(1, 16)
    pk2 = jnp.sum(kc * kc, axis=1).reshape(1, 16)
    first = (b == 0) & (nb == 0)

    @pl.when(first)
    def _():
        vs1_ref[...] = pv1
        vs2_ref[...] = pv2
        ks1_ref[...] = pk1
        ks2_ref[...] = pk2

    @pl.when(jnp.logical_not(first))
    def _():
        vs1_ref[...] += pv1
        vs2_ref[...] += pv2
        ks1_ref[...] += pk1
        ks2_ref[...] += pk2


def _stage_a(x, wvt, wk):
    return pl.pallas_call(
        _stage_a_body,
        grid=(_B, _N // _NB),
        in_specs=[
            pl.BlockSpec((1, _MD, _NB), lambda b, n: (b, 0, n)),
            pl.BlockSpec((_MD, 128), lambda b, n: (0, 0)),
            pl.BlockSpec((16, _MD), lambda b, n: (0, 0)),
        ],
        out_specs=[
            pl.BlockSpec((1, _NB, 128), lambda b, n: (b, n, 0)),
            pl.BlockSpec((1, 16, _NB), lambda b, n: (b, 0, n)),
            pl.BlockSpec((1, 128), lambda b, n: (0, 0)),
            pl.BlockSpec((1, 128), lambda b, n: (0, 0)),
            pl.BlockSpec((1, 16), lambda b, n: (0, 0)),
            pl.BlockSpec((1, 16), lambda b, n: (0, 0)),
        ],
        out_shape=[
            jax.ShapeDtypeStruct((_B, _N, 128), jnp.float32),
            jax.ShapeDtypeStruct((_B, 16, _N), jnp.float32),
            jax.ShapeDtypeStruct((1, 128), jnp.float32),
            jax.ShapeDtypeStruct((1, 128), jnp.float32),
            jax.ShapeDtypeStruct((1, 16), jnp.float32),
            jax.ShapeDtypeStruct((1, 16), jnp.float32),
        ],
    )(x, wvt, wk)


def _stage_b_body(kvk_ref, orig_ref, ps_ref, po_ref, kb_ref,
                  w_ref, idx_ref, ks1_ref, ks2_ref):
    b = pl.program_id(0)
    nb = pl.program_id(1)
    keys8 = (jnp.dot(ps_ref[...], kvk_ref[0], preferred_element_type=jnp.float32)
             + jnp.dot(po_ref[...], orig_ref[0], preferred_element_type=jnp.float32)
             + kb_ref[:, 0:1])
    ks = jnp.sum(keys8)
    ks2 = jnp.sum(keys8 * keys8)
    first = (b == 0) & (nb == 0)

    @pl.when(first)
    def _():
        ks1_ref[0, 0] = ks
        ks2_ref[0, 0] = ks2

    @pl.when(jnp.logical_not(first))
    def _():
        ks1_ref[0, 0] += ks
        ks2_ref[0, 0] += ks2

    lat = jnp.tanh(keys8)
    g = jnp.clip((lat + 1.0) * (0.5 * (_S - 1)), 0.0, _S - 1.0)
    base = jnp.clip(jnp.floor(g).astype(jnp.int32), 0, _S - 2)
    fr = g - base.astype(jnp.float32)
    fy, fx = fr[:4], fr[4:]
    by, bx = base[:4], base[4:]
    w_list, i_list = [], []
    for oy in (0, 1):
        for ox in (0, 1):
            wy = fy if oy else 1.0 - fy
            wx = fx if ox else 1.0 - fx
            w_list.append(wy * wx)
            i_list.append((by + oy) * _S + (bx + ox))
    w_ref[0] = jnp.stack(w_list, axis=1)
    idx_ref[0] = jnp.stack(i_list, axis=1)


def _stage_b(kvk, orig8, ps16, po8, kb):
    return pl.pallas_call(
        _stage_b_body,
        grid=(_B, _N // _NB),
        in_specs=[
            pl.BlockSpec((1, 16, _NB), lambda b, n: (b, 0, n)),
            pl.BlockSpec((1, 8, _NB), lambda b, n: (b, 0, n)),
            pl.BlockSpec((8, 16), lambda b, n: (0, 0)),
            pl.BlockSpec((8, 8), lambda b, n: (0, 0)),
            pl.BlockSpec((8, 128), lambda b, n: (0, 0)),
        ],
        out_specs=[
            pl.BlockSpec((1, _H, 4, _NB), lambda b, n: (b, 0, 0, n)),
            pl.BlockSpec((1, _H, 4, _NB), lambda b, n: (b, 0, 0, n)),
            pl.BlockSpec((1, 1), lambda b, n: (0, 0), memory_space=pltpu.SMEM),
            pl.BlockSpec((1, 1), lambda b, n: (0, 0), memory_space=pltpu.SMEM),
        ],
        out_shape=[
            jax.ShapeDtypeStruct((_B, _H, 4, _N), jnp.float32),
            jax.ShapeDtypeStruct((_B, _H, 4, _N), jnp.int32),
            jax.ShapeDtypeStruct((1, 1), jnp.float32),
            jax.ShapeDtypeStruct((1, 1), jnp.float32),
        ],
    )(kvk, orig8, ps16, po8, kb)


def _splat_sc(vt, vsc, vbi, w_all, idx_all, zinit):
    @functools.partial(
        pl.kernel,
        out_type=jax.ShapeDtypeStruct((_B, _G, 128), jnp.float32),
        mesh=_sc_mesh(),
        compiler_params=_SC_PARAMS,
        scratch_types=[
            pltpu.VMEM((_G, 16), jnp.float32),
            pltpu.VMEM((2, _NPC, 16), jnp.float32),
            pltpu.VMEM((2, 4, _NPC), jnp.float32),
            pltpu.VMEM((2, 4, _NPC), jnp.int32),
            pltpu.VMEM((16,), jnp.float32),
            pltpu.VMEM((16,), jnp.float32),
            pltpu.SemaphoreType.DMA((2,)),
            pltpu.SemaphoreType.DMA((2,)),
            pltpu.SemaphoreType.DMA((2,)),
        ],
    )
    def k(vt_hbm, vsc_hbm, vbi_hbm, w_hbm, idx_hbm, zer_hbm, z_hbm,
          grid_v, vals_v, w_v, i_v, sc_v, bi_v, sem_v, sem_w, sem_i):
        wid = lax.axis_index("s") * 2 + lax.axis_index("c")
        b = wid // 8
        r = wid % 8
        h = r // 2
        half = r % 2
        ch0 = h * 32 + half * 16

        def start(ci, s):
            p0 = ci * _NPC
            pltpu.async_copy(vt_hbm.at[b, pl.ds(p0, _NPC), pl.ds(ch0, 16)],
                             vals_v.at[s], sem_v.at[s])
            pltpu.async_copy(w_hbm.at[b, h, :, pl.ds(p0, _NPC)],
                             w_v.at[s], sem_w.at[s])
            pltpu.async_copy(idx_hbm.at[b, h, :, pl.ds(p0, _NPC)],
                             i_v.at[s], sem_i.at[s])

        def drain(s):
            pltpu.make_async_copy(vt_hbm.at[b, pl.ds(0, _NPC), pl.ds(ch0, 16)],
                                  vals_v.at[s], sem_v.at[s]).wait()
            pltpu.make_async_copy(w_hbm.at[b, h, :, pl.ds(0, _NPC)],
                                  w_v.at[s], sem_w.at[s]).wait()
            pltpu.make_async_copy(idx_hbm.at[b, h, :, pl.ds(0, _NPC)],
                                  i_v.at[s], sem_i.at[s]).wait()

        start(0, 0)
        pltpu.sync_copy(zer_hbm, grid_v)
        pltpu.sync_copy(vsc_hbm.at[pl.ds(ch0, 16)], sc_v)
        pltpu.sync_copy(vbi_hbm.at[pl.ds(ch0, 16)], bi_v)
        sc = sc_v[...]
        bi = bi_v[...]
        nchunk = _N // _NPC

        def chunk(ci, carry):
            s = lax.rem(ci, 2)

            @pl.when(ci + 1 < nchunk)
            def _():
                start(ci + 1, lax.rem(ci + 1, 2))

            drain(s)

            @plsc.parallel_loop(0, _NPC // 16, 1, unroll=8)
            def blk16(gq):
                q0 = gq * 16
                wrows = [w_v[s, c, pl.ds(q0, 16)] for c in range(4)]
                irows = [i_v[s, c, pl.ds(q0, 16)] for c in range(4)]
                for j in range(16):
                    vn = vals_v[s, q0 + j, :] * sc + bi
                    for c in range(4):
                        plsc.addupdate(grid_v.at[irows[c][j]],
                                       vn * wrows[c][j])

            return carry

        lax.fori_loop(0, nchunk, chunk, 0)
        pltpu.sync_copy(grid_v, z_hbm.at[b, :, pl.ds(ch0, 16)])

    return k(vt, vsc, vbi, w_all, idx_all, zinit)


def _stage_d_body(z_ref, wblk_ref, m_ref, zc_ref, occ_ref):
    b = pl.program_id(0)
    z2 = z_ref[0]
    cnt = jnp.sum((jnp.abs(z2) > 1e-9).astype(jnp.float32))

    @pl.when(b == 0)
    def _():
        occ_ref[0, 0] = cnt

    @pl.when(b != 0)
    def _():
        occ_ref[0, 0] += cnt

    acc = None
    for t in range(9):
        sy, sx = t // 3 - 1, t % 3 - 1
        kk = sy * _S + sx
        zr = z2 if kk == 0 else jnp.roll(z2, -kk, axis=0)
        zm = zr * m_ref[:, t:t + 1]
        d = jnp.dot(zm, wblk_ref[t], preferred_element_type=jnp.float32)
        acc = d if acc is None else acc + d
    zc_ref[0] = acc


def _stage_d(z, wblk, masks):
    return pl.pallas_call(
        _stage_d_body,
        grid=(_B,),
        in_specs=[
            pl.BlockSpec((1, _G, 128), lambda b: (b, 0, 0)),
            pl.BlockSpec((9, 128, 128), lambda b: (0, 0, 0)),
            pl.BlockSpec((_G, 16), lambda b: (0, 0)),
        ],
        out_specs=[
            pl.BlockSpec((1, _G, 128), lambda b: (b, 0, 0)),
            pl.BlockSpec((1, 1), lambda b: (0, 0), memory_space=pltpu.SMEM),
        ],
        out_shape=[
            jax.ShapeDtypeStruct((_B, _G, 128), jnp.float32),
            jax.ShapeDtypeStruct((1, 1), jnp.float32),
        ],
    )(z, wblk, masks)


def _slice_sc(zc, w_all, idx_all):
    @functools.partial(
        pl.kernel,
        out_type=(jax.ShapeDtypeStruct((_B, _N, 128), jnp.float32),
                  jax.ShapeDtypeStruct((32, 2, 16), jnp.float32)),
        mesh=_sc_mesh(),
        compiler_params=_SC_PARAMS,
        scratch_types=[
            pltpu.VMEM((_G, 16), jnp.float32),
            pltpu.VMEM((2, _NPC, 16), jnp.float32),
            pltpu.VMEM((2, 4, _NPC), jnp.float32),
            pltpu.VMEM((2, 4, _NPC), jnp.int32),
            pltpu.VMEM((2, 16), jnp.float32),
            pltpu.SemaphoreType.DMA((2,)),
            pltpu.SemaphoreType.DMA((2,)),
            pltpu.SemaphoreType.DMA((2,)),
        ],
    )
    def k(zc_hbm, w_hbm, idx_hbm, out_hbm, st_hbm,
          grid_v, obuf_v, w_v, i_v, st_v, sem_w, sem_i, sem_o):
        wid = lax.axis_index("s") * 2 + lax.axis_index("c")
        b = wid // 8
        r = wid % 8
        h = r // 2
        half = r % 2
        ch0 = h * 32 + half * 16

        def start(ci, s):
            p0 = ci * _NPC
            pltpu.async_copy(w_hbm.at[b, h, :, pl.ds(p0, _NPC)],
                             w_v.at[s], sem_w.at[s])
            pltpu.async_copy(idx_hbm.at[b, h, :, pl.ds(p0, _NPC)],
                             i_v.at[s], sem_i.at[s])

        def drain(s):
            pltpu.make_async_copy(w_hbm.at[b, h, :, pl.ds(0, _NPC)],
                                  w_v.at[s], sem_w.at[s]).wait()
            pltpu.make_async_copy(idx_hbm.at[b, h, :, pl.ds(0, _NPC)],
                                  i_v.at[s], sem_i.at[s]).wait()

        start(0, 0)
        pltpu.sync_copy(zc_hbm.at[b, :, pl.ds(ch0, 16)], grid_v)
        zero16 = jnp.zeros((16,), jnp.float32)
        nchunk = _N // _NPC

        def chunk(ci, carry):
            s = lax.rem(ci, 2)

            @pl.when(ci + 1 < nchunk)
            def _():
                start(ci + 1, lax.rem(ci + 1, 2))

            drain(s)

            @pl.when(ci >= 2)
            def _():
                pltpu.make_async_copy(
                    obuf_v.at[s],
                    out_hbm.at[b, pl.ds(0, _NPC), pl.ds(ch0, 16)],
                    sem_o.at[s]).wait()

            @plsc.parallel_loop(0, _NPC // 16, 1, unroll=8, carry=carry)
            def blk16(gq, cr):
                s1, s2 = cr
                q0 = gq * 16
                wrows = [w_v[s, c, pl.ds(q0, 16)] for c in range(4)]
                irows = [i_v[s, c, pl.ds(q0, 16)] for c in range(4)]
                for j in range(16):
                    acc = grid_v[irows[0][j], :] * wrows[0][j]
                    for c in range(1, 4):
                        acc = acc + grid_v[irows[c][j], :] * wrows[c][j]
                    obuf_v[s, q0 + j, :] = acc
                    s1 = s1 + acc
                    s2 = s2 + acc * acc
                return (s1, s2)

            p0 = ci * _NPC
            pltpu.async_copy(obuf_v.at[s],
                             out_hbm.at[b, pl.ds(p0, _NPC), pl.ds(ch0, 16)],
                             sem_o.at[s])
            return blk16

        carry = lax.fori_loop(0, nchunk, chunk, (zero16, zero16))
        s1, s2 = carry
        pltpu.make_async_copy(obuf_v.at[0],
                              out_hbm.at[b, pl.ds(0, _NPC), pl.ds(ch0, 16)],
                              sem_o.at[0]).wait()
        pltpu.make_async_copy(obuf_v.at[1],
                              out_hbm.at[b, pl.ds(0, _NPC), pl.ds(ch0, 16)],
                              sem_o.at[1]).wait()
        st_v[0, :] = s1
        st_v[1, :] = s2
        pltpu.sync_copy(st_v, st_hbm.at[wid])

    return k(zc, w_all, idx_all)


def _stage_f_body(x_ref, sc_ref, bi_ref, o_ref):
    x = x_ref[0]
    y = jnp.maximum(x * sc_ref[0:1, :] + bi_ref[0:1, :], 0.0)
    o_ref[0] = y.T


def _stage_f(xt, sc1, bi1):
    return pl.pallas_call(
        _stage_f_body,
        grid=(_B, _N // _NB),
        in_specs=[
            pl.BlockSpec((1, _NB, 128), lambda b, n: (b, n, 0)),
            pl.BlockSpec((1, 128), lambda b, n: (0, 0)),
            pl.BlockSpec((1, 128), lambda b, n: (0, 0)),
        ],
        out_specs=[pl.BlockSpec((1, 128, _NB), lambda b, n: (b, 0, n))],
        out_shape=[jax.ShapeDtypeStruct((_B, 128, _N), jnp.float32)],
    )(xt, sc1, bi1)


def kernel(input, orig_pcd, W_kv, kbn_gamma, kbn_beta, vbn_gamma, vbn_beta,
           P, conv_w, conv_b, abn_gamma, abn_beta):
    f32 = jnp.float32
    wvt = jnp.swapaxes(W_kv[12:140], 0, 1)
    wk = jnp.pad(W_kv[:12], ((0, 4), (0, 0)))
    orig8 = jnp.pad(orig_pcd, ((0, 0), (0, 5), (0, 0)))

    pm = []
    for d in range(2):
        rows = []
        for h in range(4):
            row = jnp.zeros((12,), f32).at[3 * h:3 * h + 3].set(P[h, d, :])
            rows.append(row)
        pm.append(jnp.stack(rows))
    Pmat = jnp.concatenate(pm, axis=0)
    Po = jnp.concatenate([P[:, 0, :], P[:, 1, :]], axis=0)

    vt, kvk, vs1, vs2, ks1a, ks2a = _stage_a(input, wvt, wk)

    vmean = vs1[0] / _CNT
    vvar = vs2[0] / _CNT - vmean * vmean
    vsc = vbn_gamma * lax.rsqrt(vvar + 1e-5)
    vbi = vbn_beta - vmean * vsc

    kmean = ks1a[0, :12] / _CNT
    kvar = ks2a[0, :12] / _CNT - kmean * kmean
    ksc = kbn_gamma * lax.rsqrt(kvar + 1e-5)
    kbi = kbn_beta - kmean * ksc

    ps16 = jnp.pad(Pmat * ksc[None, :], ((0, 0), (0, 4)))
    po8 = jnp.pad(Po, ((0, 0), (0, 5)))
    kb = jnp.broadcast_to((Pmat @ kbi).reshape(8, 1), (8, 128))

    w_all, idx_all, ks1, ks2 = _stage_b(kvk, orig8, ps16, po8, kb)

    zinit = jnp.zeros((_G, 16), f32)
    z = _splat_sc(vt, vsc, vbi, w_all, idx_all, zinit)

    wblk_rows = []
    for t in range(9):
        dy, dx = t // 3, t % 3
        taps = [conv_w[h * 32:(h + 1) * 32, :, dy, dx] for h in range(4)]
        wblk_rows.append(jax.scipy.linalg.block_diag(*taps).T)
    wblk = jnp.stack(wblk_rows)
    cells = jnp.arange(_G, dtype=jnp.int32)
    cy, cx = cells // _S, cells % _S
    mcols = []
    for t in range(9):
        sy, sx = t // 3 - 1, t % 3 - 1
        ok = ((cy + sy >= 0) & (cy + sy < _S) & (cx + sx >= 0) & (cx + sx < _S))
        mcols.append(ok.astype(f32))
    masks = jnp.pad(jnp.stack(mcols, axis=1), ((0, 0), (0, 7)))

    zc, occ_raw = _stage_d(z, wblk, masks)

    sliced_t, st = _slice_sc(zc, w_all, idx_all)

    stv = st.reshape(_B, 8, 2, 16).sum(axis=0)
    t1o = stv[:, 0, :].reshape(128)
    t2o = stv[:, 1, :].reshape(128)
    t1 = t1o + conv_b * _CNT
    t2 = t2o + 2.0 * conv_b * t1o + conv_b * conv_b * _CNT
    m2 = t1 / _CNT
    v2 = t2 / _CNT - m2 * m2
    sc2 = abn_gamma * lax.rsqrt(v2 + 1e-5)
    bi2 = abn_beta - m2 * sc2 + conv_b * sc2
    sc1 = sc2.reshape(1, 128)
    bi1 = bi2.reshape(1, 128)

    result, = _stage_f(sliced_t, sc1, bi1)

    occ = occ_raw[0, 0] / (_B * _F * _H)
    km = ks1[0, 0] / (_B * 8 * _N)
    kvv = ks2[0, 0] / (_B * 8 * _N) - km * km
    return result, occ, km, kvv


# speedup vs baseline: 1.0249x; 1.0249x over previous
import functools

import jax
import jax.numpy as jnp
from jax import lax
from jax.experimental import pallas as pl
from jax.experimental.pallas import tpu as pltpu
from jax.experimental.pallas import tpu_sc as plsc

_B = 4
_MD = 128
_N = 8192
_H = 4
_F = 32
_S = 64
_NB = 2048
_NPC = 1024
_G = _S * _S
_CNT = _B * _N

_SC_PARAMS = pltpu.CompilerParams(use_tc_tiling_on_sc=False,
                                  needs_layout_passes=False)


def _sc_mesh():
    return plsc.VectorSubcoreMesh(core_axis_name="c", subcore_axis_name="s",
                                  num_cores=2, num_subcores=16)


def _stage_a_body(x_ref, wvt_ref, wk_ref, vt_ref, kvk_ref,
                  vs1_ref, vs2_ref, ks1_ref, ks2_ref):
    b = pl.program_id(0)
    nb = pl.program_id(1)
    x = x_ref[0]
    vtc = lax.dot_general(x, wvt_ref[...], (((0,), (0,)), ((), ())),
                          preferred_element_type=jnp.float32)
    vt_ref[0] = vtc
    kc = jnp.dot(wk_ref[...], x, preferred_element_type=jnp.float32)
    kvk_ref[0] = kc
    pv1 = jnp.sum(vtc, axis=0).reshape(1, 128)
    pv2 = jnp.sum(vtc * vtc, axis=0).reshape(1, 128)
    pk1 = jnp.sum(kc, axis=1).reshape(1, 16)
    pk2 = jnp.sum(kc * kc, axis=1).reshape(1, 16)
    first = (b == 0) & (nb == 0)

    @pl.when(first)
    def _():
        vs1_ref[...] = pv1
        vs2_ref[...] = pv2
        ks1_ref[...] = pk1
        ks2_ref[...] = pk2

    @pl.when(jnp.logical_not(first))
    def _():
        vs1_ref[...] += pv1
        vs2_ref[...] += pv2
        ks1_ref[...] += pk1
        ks2_ref[...] += pk2


def _stage_a(x, wvt, wk):
    return pl.pallas_call(
        _stage_a_body,
        grid=(_B, _N // _NB),
        in_specs=[
            pl.BlockSpec((1, _MD, _NB), lambda b, n: (b, 0, n)),
            pl.BlockSpec((_MD, 128), lambda b, n: (0, 0)),
            pl.BlockSpec((16, _MD), lambda b, n: (0, 0)),
        ],
        out_specs=[
            pl.BlockSpec((1, _NB, 128), lambda b, n: (b, n, 0)),
            pl.BlockSpec((1, 16, _NB), lambda b, n: (b, 0, n)),
            pl.BlockSpec((1, 128), lambda b, n: (0, 0)),
            pl.BlockSpec((1, 128), lambda b, n: (0, 0)),
            pl.BlockSpec((1, 16), lambda b, n: (0, 0)),
            pl.BlockSpec((1, 16), lambda b, n: (0, 0)),
        ],
        out_shape=[
            jax.ShapeDtypeStruct((_B, _N, 128), jnp.float32),
            jax.ShapeDtypeStruct((_B, 16, _N), jnp.float32),
            jax.ShapeDtypeStruct((1, 128), jnp.float32),
            jax.ShapeDtypeStruct((1, 128), jnp.float32),
            jax.ShapeDtypeStruct((1, 16), jnp.float32),
            jax.ShapeDtypeStruct((1, 16), jnp.float32),
        ],
    )(x, wvt, wk)


def _stage_b_body(kvk_ref, orig_ref, ps_ref, po_ref, kb_ref,
                  w_ref, idx_ref, ks1_ref, ks2_ref):
    b = pl.program_id(0)
    nb = pl.program_id(1)
    keys8 = (jnp.dot(ps_ref[...], kvk_ref[0], preferred_element_type=jnp.float32)
             + jnp.dot(po_ref[...], orig_ref[0], preferred_element_type=jnp.float32)
             + kb_ref[:, 0:1])
    ks = jnp.sum(keys8)
    ks2 = jnp.sum(keys8 * keys8)
    first = (b == 0) & (nb == 0)

    @pl.when(first)
    def _():
        ks1_ref[0, 0] = ks
        ks2_ref[0, 0] = ks2

    @pl.when(jnp.logical_not(first))
    def _():
        ks1_ref[0, 0] += ks
        ks2_ref[0, 0] += ks2

    lat = jnp.tanh(keys8)
    g = jnp.clip((lat + 1.0) * (0.5 * (_S - 1)), 0.0, _S - 1.0)
    base = jnp.clip(jnp.floor(g).astype(jnp.int32), 0, _S - 2)
    fr = g - base.astype(jnp.float32)
    fy, fx = fr[:4], fr[4:]
    by, bx = base[:4], base[4:]
    w_list, i_list = [], []
    for oy in (0, 1):
        for ox in (0, 1):
            wy = fy if oy else 1.0 - fy
            wx = fx if ox else 1.0 - fx
            w_list.append(wy * wx)
            i_list.append((by + oy) * _S + (bx + ox))
    w_ref[0] = jnp.stack(w_list, axis=1)
    idx_ref[0] = jnp.stack(i_list, axis=1)


def _stage_b(kvk, orig8, ps16, po8, kb):
    return pl.pallas_call(
        _stage_b_body,
        grid=(_B, _N // _NB),
        in_specs=[
            pl.BlockSpec((1, 16, _NB), lambda b, n: (b, 0, n)),
            pl.BlockSpec((1, 8, _NB), lambda b, n: (b, 0, n)),
            pl.BlockSpec((8, 16), lambda b, n: (0, 0)),
            pl.BlockSpec((8, 8), lambda b, n: (0, 0)),
            pl.BlockSpec((8, 128), lambda b, n: (0, 0)),
        ],
        out_specs=[
            pl.BlockSpec((1, _H, 4, _NB), lambda b, n: (b, 0, 0, n)),
            pl.BlockSpec((1, _H, 4, _NB), lambda b, n: (b, 0, 0, n)),
            pl.BlockSpec((1, 1), lambda b, n: (0, 0), memory_space=pltpu.SMEM),
            pl.BlockSpec((1, 1), lambda b, n: (0, 0), memory_space=pltpu.SMEM),
        ],
        out_shape=[
            jax.ShapeDtypeStruct((_B, _H, 4, _N), jnp.float32),
            jax.ShapeDtypeStruct((_B, _H, 4, _N), jnp.int32),
            jax.ShapeDtypeStruct((1, 1), jnp.float32),
            jax.ShapeDtypeStruct((1, 1), jnp.float32),
        ],
    )(kvk, orig8, ps16, po8, kb)


def _splat_sc(vt, vsc, vbi, w_all, idx_all, zinit):
    @functools.partial(
        pl.kernel,
        out_type=jax.ShapeDtypeStruct((_B, _G, 128), jnp.float32),
        mesh=_sc_mesh(),
        compiler_params=_SC_PARAMS,
        scratch_types=[
            pltpu.VMEM((_G, 16), jnp.float32),
            pltpu.VMEM((2, _NPC, 16), jnp.float32),
            pltpu.VMEM((2, 4, _NPC), jnp.float32),
            pltpu.VMEM((2, 4, _NPC), jnp.int32),
            pltpu.VMEM((16,), jnp.float32),
            pltpu.VMEM((16,), jnp.float32),
            pltpu.SemaphoreType.DMA((2,)),
            pltpu.SemaphoreType.DMA((2,)),
            pltpu.SemaphoreType.DMA((2,)),
        ],
    )
    def k(vt_hbm, vsc_hbm, vbi_hbm, w_hbm, idx_hbm, zer_hbm, z_hbm,
          grid_v, vals_v, w_v, i_v, sc_v, bi_v, sem_v, sem_w, sem_i):
        wid = lax.axis_index("s") * 2 + lax.axis_index("c")
        b = wid // 8
        r = wid % 8
        h = r // 2
        half = r % 2
        ch0 = h * 32 + half * 16

        def start(ci, s):
            p0 = ci * _NPC
            pltpu.async_copy(vt_hbm.at[b, pl.ds(p0, _NPC), pl.ds(ch0, 16)],
                             vals_v.at[s], sem_v.at[s])
            pltpu.async_copy(w_hbm.at[b, h, :, pl.ds(p0, _NPC)],
                             w_v.at[s], sem_w.at[s])
            pltpu.async_copy(idx_hbm.at[b, h, :, pl.ds(p0, _NPC)],
                             i_v.at[s], sem_i.at[s])

        def drain(s):
            pltpu.make_async_copy(vt_hbm.at[b, pl.ds(0, _NPC), pl.ds(ch0, 16)],
                                  vals_v.at[s], sem_v.at[s]).wait()
            pltpu.make_async_copy(w_hbm.at[b, h, :, pl.ds(0, _NPC)],
                                  w_v.at[s], sem_w.at[s]).wait()
            pltpu.make_async_copy(idx_hbm.at[b, h, :, pl.ds(0, _NPC)],
                                  i_v.at[s], sem_i.at[s]).wait()

        start(0, 0)
        pltpu.sync_copy(zer_hbm, grid_v)
        pltpu.sync_copy(vsc_hbm.at[pl.ds(ch0, 16)], sc_v)
        pltpu.sync_copy(vbi_hbm.at[pl.ds(ch0, 16)], bi_v)
        sc = sc_v[...]
        bi = bi_v[...]
        nchunk = _N // _NPC

        def chunk(ci, carry):
            s = lax.rem(ci, 2)

            @pl.when(ci + 1 < nchunk)
            def _():
                start(ci + 1, lax.rem(ci + 1, 2))

            drain(s)

            @plsc.parallel_loop(0, _NPC // 16, 1, unroll=4)
            def blk16(gq):
                q0 = gq * 16
                wrows = [w_v[s, c, pl.ds(q0, 16)] for c in range(4)]
                irows = [i_v[s, c, pl.ds(q0, 16)] for c in range(4)]
                for j in range(16):
                    vn = vals_v[s, q0 + j, :] * sc + bi
                    for c in range(4):
                        plsc.addupdate(grid_v.at[irows[c][j]],
                                       vn * wrows[c][j])

            return carry

        lax.fori_loop(0, nchunk, chunk, 0)
        pltpu.sync_copy(grid_v, z_hbm.at[b, :, pl.ds(ch0, 16)])

    return k(vt, vsc, vbi, w_all, idx_all, zinit)


def _stage_d_body(z_ref, wblk_ref, m_ref, zc_ref, occ_ref):
    b = pl.program_id(0)
    z2 = z_ref[0]
    cnt = jnp.sum((jnp.abs(z2) > 1e-9).astype(jnp.float32))

    @pl.when(b == 0)
    def _():
        occ_ref[0, 0] = cnt

    @pl.when(b != 0)
    def _():
        occ_ref[0, 0] += cnt

    acc = None
    for t in range(9):
        sy, sx = t // 3 - 1, t % 3 - 1
        kk = sy * _S + sx
        zr = z2 if kk == 0 else jnp.roll(z2, -kk, axis=0)
        zm = zr * m_ref[:, t:t + 1]
        d = jnp.dot(zm, wblk_ref[t], preferred_element_type=jnp.float32)
        acc = d if acc is None else acc + d
    zc_ref[0] = acc


def _stage_d(z, wblk, masks):
    return pl.pallas_call(
        _stage_d_body,
        grid=(_B,),
        in_specs=[
            pl.BlockSpec((1, _G, 128), lambda b: (b, 0, 0)),
            pl.BlockSpec((9, 128, 128), lambda b: (0, 0, 0)),
            pl.BlockSpec((_G, 16), lambda b: (0, 0)),
        ],
        out_specs=[
            pl.BlockSpec((1, _G, 128), lambda b: (b, 0, 0)),
            pl.BlockSpec((1, 1), lambda b: (0, 0), memory_space=pltpu.SMEM),
        ],
        out_shape=[
            jax.ShapeDtypeStruct((_B, _G, 128), jnp.float32),
            jax.ShapeDtypeStruct((1, 1), jnp.float32),
        ],
    )(z, wblk, masks)


def _slice_sc(zc, w_all, idx_all):
    @functools.partial(
        pl.kernel,
        out_type=(jax.ShapeDtypeStruct((_B, _N, 128), jnp.float32),
                  jax.ShapeDtypeStruct((32, 2, 16), jnp.float32)),
        mesh=_sc_mesh(),
        compiler_params=_SC_PARAMS,
        scratch_types=[
            pltpu.VMEM((_G, 16), jnp.float32),
            pltpu.VMEM((2, _NPC, 16), jnp.float32),
            pltpu.VMEM((2, 4, _NPC), jnp.float32),
            pltpu.VMEM((2, 4, _NPC), jnp.int32),
            pltpu.VMEM((2, 16), jnp.float32),
            pltpu.SemaphoreType.DMA((2,)),
            pltpu.SemaphoreType.DMA((2,)),
            pltpu.SemaphoreType.DMA((2,)),
        ],
    )
    def k(zc_hbm, w_hbm, idx_hbm, out_hbm, st_hbm,
          grid_v, obuf_v, w_v, i_v, st_v, sem_w, sem_i, sem_o):
        wid = lax.axis_index("s") * 2 + lax.axis_index("c")
        b = wid // 8
        r = wid % 8
        h = r // 2
        half = r % 2
        ch0 = h * 32 + half * 16

        def start(ci, s):
            p0 = ci * _NPC
            pltpu.async_copy(w_hbm.at[b, h, :, pl.ds(p0, _NPC)],
                             w_v.at[s], sem_w.at[s])
            pltpu.async_copy(idx_hbm.at[b, h, :, pl.ds(p0, _NPC)],
                             i_v.at[s], sem_i.at[s])

        def drain(s):
            pltpu.make_async_copy(w_hbm.at[b, h, :, pl.ds(0, _NPC)],
                                  w_v.at[s], sem_w.at[s]).wait()
            pltpu.make_async_copy(idx_hbm.at[b, h, :, pl.ds(0, _NPC)],
                                  i_v.at[s], sem_i.at[s]).wait()

        start(0, 0)
        pltpu.sync_copy(zc_hbm.at[b, :, pl.ds(ch0, 16)], grid_v)
        zero16 = jnp.zeros((16,), jnp.float32)
        nchunk = _N // _NPC

        def chunk(ci, carry):
            s = lax.rem(ci, 2)

            @pl.when(ci + 1 < nchunk)
            def _():
                start(ci + 1, lax.rem(ci + 1, 2))

            drain(s)

            @pl.when(ci >= 2)
            def _():
                pltpu.make_async_copy(
                    obuf_v.at[s],
                    out_hbm.at[b, pl.ds(0, _NPC), pl.ds(ch0, 16)],
                    sem_o.at[s]).wait()

            @plsc.parallel_loop(0, _NPC // 16, 1, unroll=4, carry=carry)
            def blk16(gq, cr):
                s1, s2 = cr
                q0 = gq * 16
                wrows = [w_v[s, c, pl.ds(q0, 16)] for c in range(4)]
                irows = [i_v[s, c, pl.ds(q0, 16)] for c in range(4)]
                for j in range(16):
                    acc = grid_v[irows[0][j], :] * wrows[0][j]
                    for c in range(1, 4):
                        acc = acc + grid_v[irows[c][j], :] * wrows[c][j]
                    obuf_v[s, q0 + j, :] = acc
                    s1 = s1 + acc
                    s2 = s2 + acc * acc
                return (s1, s2)

            p0 = ci * _NPC
            pltpu.async_copy(obuf_v.at[s],
                             out_hbm.at[b, pl.ds(p0, _NPC), pl.ds(ch0, 16)],
                             sem_o.at[s])
            return blk16

        carry = lax.fori_loop(0, nchunk, chunk, (zero16, zero16))
        s1, s2 = carry
        pltpu.make_async_copy(obuf_v.at[0],
                              out_hbm.at[b, pl.ds(0, _NPC), pl.ds(ch0, 16)],
                              sem_o.at[0]).wait()
        pltpu.make_async_copy(obuf_v.at[1],
                              out_hbm.at[b, pl.ds(0, _NPC), pl.ds(ch0, 16)],
                              sem_o.at[1]).wait()
        st_v[0, :] = s1
        st_v[1, :] = s2
        pltpu.sync_copy(st_v, st_hbm.at[wid])

    return k(zc, w_all, idx_all)


def _stage_f_body(x_ref, sc_ref, bi_ref, o_ref):
    x = x_ref[0]
    y = jnp.maximum(x * sc_ref[0:1, :] + bi_ref[0:1, :], 0.0)
    o_ref[0] = y.T


def _stage_f(xt, sc1, bi1):
    return pl.pallas_call(
        _stage_f_body,
        grid=(_B, _N // _NB),
        in_specs=[
            pl.BlockSpec((1, _NB, 128), lambda b, n: (b, n, 0)),
            pl.BlockSpec((1, 128), lambda b, n: (0, 0)),
            pl.BlockSpec((1, 128), lambda b, n: (0, 0)),
        ],
        out_specs=[pl.BlockSpec((1, 128, _NB), lambda b, n: (b, 0, n))],
        out_shape=[jax.ShapeDtypeStruct((_B, 128, _N), jnp.float32)],
    )(xt, sc1, bi1)


def kernel(input, orig_pcd, W_kv, kbn_gamma, kbn_beta, vbn_gamma, vbn_beta,
           P, conv_w, conv_b, abn_gamma, abn_beta):
    f32 = jnp.float32
    wvt = jnp.swapaxes(W_kv[12:140], 0, 1)
    wk = jnp.pad(W_kv[:12], ((0, 4), (0, 0)))
    orig8 = jnp.pad(orig_pcd, ((0, 0), (0, 5), (0, 0)))

    pm = []
    for d in range(2):
        rows = []
        for h in range(4):
            row = jnp.zeros((12,), f32).at[3 * h:3 * h + 3].set(P[h, d, :])
            rows.append(row)
        pm.append(jnp.stack(rows))
    Pmat = jnp.concatenate(pm, axis=0)
    Po = jnp.concatenate([P[:, 0, :], P[:, 1, :]], axis=0)

    vt, kvk, vs1, vs2, ks1a, ks2a = _stage_a(input, wvt, wk)

    vmean = vs1[0] / _CNT
    vvar = vs2[0] / _CNT - vmean * vmean
    vsc = vbn_gamma * lax.rsqrt(vvar + 1e-5)
    vbi = vbn_beta - vmean * vsc

    kmean = ks1a[0, :12] / _CNT
    kvar = ks2a[0, :12] / _CNT - kmean * kmean
    ksc = kbn_gamma * lax.rsqrt(kvar + 1e-5)
    kbi = kbn_beta - kmean * ksc

    ps16 = jnp.pad(Pmat * ksc[None, :], ((0, 0), (0, 4)))
    po8 = jnp.pad(Po, ((0, 0), (0, 5)))
    kb = jnp.broadcast_to((Pmat @ kbi).reshape(8, 1), (8, 128))

    w_all, idx_all, ks1, ks2 = _stage_b(kvk, orig8, ps16, po8, kb)

    zinit = jnp.zeros((_G, 16), f32)
    z = _splat_sc(vt, vsc, vbi, w_all, idx_all, zinit)

    wblk_rows = []
    for t in range(9):
        dy, dx = t // 3, t % 3
        taps = [conv_w[h * 32:(h + 1) * 32, :, dy, dx] for h in range(4)]
        wblk_rows.append(jax.scipy.linalg.block_diag(*taps).T)
    wblk = jnp.stack(wblk_rows)
    cells = jnp.arange(_G, dtype=jnp.int32)
    cy, cx = cells // _S, cells % _S
    mcols = []
    for t in range(9):
        sy, sx = t // 3 - 1, t % 3 - 1
        ok = ((cy + sy >= 0) & (cy + sy < _S) & (cx + sx >= 0) & (cx + sx < _S))
        mcols.append(ok.astype(f32))
    masks = jnp.pad(jnp.stack(mcols, axis=1), ((0, 0), (0, 7)))

    zc, occ_raw = _stage_d(z, wblk, masks)

    sliced_t, st = _slice_sc(zc, w_all, idx_all)

    stv = st.reshape(_B, 8, 2, 16).sum(axis=0)
    t1o = stv[:, 0, :].reshape(128)
    t2o = stv[:, 1, :].reshape(128)
    t1 = t1o + conv_b * _CNT
    t2 = t2o + 2.0 * conv_b * t1o + conv_b * conv_b * _CNT
    m2 = t1 / _CNT
    v2 = t2 / _CNT - m2 * m2
    sc2 = abn_gamma * lax.rsqrt(v2 + 1e-5)
    bi2 = abn_beta - m2 * sc2 + conv_b * sc2
    sc1 = sc2.reshape(1, 128)
    bi1 = bi2.reshape(1, 128)

    result, = _stage_f(sliced_t, sc1, bi1)

    occ = occ_raw[0, 0] / (_B * _F * _H)
    km = ks1[0, 0] / (_B * 8 * _N)
    kvv = ks2[0, 0] / (_B * 8 * _N) - km * km
    return result, occ, km, kvv
